# baseline (speedup 1.0000x reference)
import functools

import jax
import jax.numpy as jnp
from jax import lax
from jax.experimental import pallas as pl
from jax.experimental.pallas import tpu as pltpu
from jax.experimental.pallas import tpu_sc as plsc

_D = 64
_B = 16384
_NC = 2
_NS = 16
_NW = _NC * _NS
_ROWS_PER_W = _B // _NW
_CHUNK = 256
_K = _ROWS_PER_W // _CHUNK
_GROUPS = _CHUNK // 16
_SG = 11
_NSTREAM = _SG * 16
_mesh = plsc.VectorSubcoreMesh(core_axis_name="c", subcore_axis_name="s")


@functools.partial(
    pl.kernel,
    out_type=(
        jax.ShapeDtypeStruct((_B, _D), jnp.float32),
        jax.ShapeDtypeStruct((_B, _D), jnp.float32),
    ),
    mesh=_mesh,
    scratch_types=[
        pltpu.VMEM((_ROWS_PER_W,), jnp.int32),
        pltpu.VMEM((_ROWS_PER_W,), jnp.int32),
        pltpu.VMEM((_NSTREAM, _D), jnp.float32),
        pltpu.VMEM((_NSTREAM, _D), jnp.float32),
        pltpu.SemaphoreType.DMA,
        pltpu.SemaphoreType.DMA,
        pltpu.SemaphoreType.DMA,
    ],
)
def _od_gather(table, ori, dest, o_out, d_out, oidx_v, didx_v, obuf, dbuf,
               sem_o, sem_d, sem_w):
    wid = lax.axis_index("s") * _NC + lax.axis_index("c")
    row0 = wid * _ROWS_PER_W
    pltpu.sync_copy(ori.at[pl.ds(row0, _ROWS_PER_W)], oidx_v)
    pltpu.sync_copy(dest.at[pl.ds(row0, _ROWS_PER_W)], didx_v)

    def chunk_body(c, _):
        cbase = c * _CHUNK

        def group_body(g, _):
            ovec = oidx_v[pl.ds(cbase + g * 16, 16)]
            dvec = didx_v[pl.ds(cbase + g * 16, 16)]

            def do_stream():
                for l in range(16):
                    pltpu.async_copy(
                        table.at[pl.ds(ovec[l], 1)],
                        obuf.at[pl.ds(g * 16 + l, 1)],
                        sem_o,
                    )
                    pltpu.async_copy(
                        table.at[pl.ds(dvec[l], 1)],
                        dbuf.at[pl.ds(g * 16 + l, 1)],
                        sem_d,
                    )

            def do_dma():
                out_base = row0 + cbase + g * 16
                for l in range(16):
                    pltpu.async_copy(
                        table.at[pl.ds(ovec[l], 1)],
                        o_out.at[pl.ds(out_base + l, 1)],
                        sem_w,
                    )
                    pltpu.async_copy(
                        table.at[pl.ds(dvec[l], 1)],
                        d_out.at[pl.ds(out_base + l, 1)],
                        sem_w,
                    )

            lax.cond(g < _SG, do_stream, do_dma)
            return ()

        lax.fori_loop(0, _GROUPS, group_body, ())
        pltpu.make_async_copy(table.at[pl.ds(0, _NSTREAM)], obuf, sem_o).wait()
        pltpu.make_async_copy(table.at[pl.ds(0, _NSTREAM)], dbuf, sem_d).wait()
        pltpu.sync_copy(obuf, o_out.at[pl.ds(row0 + cbase, _NSTREAM)])
        pltpu.sync_copy(dbuf, d_out.at[pl.ds(row0 + cbase, _NSTREAM)])
        return ()

    lax.fori_loop(0, _K, chunk_body, ())
    n_dma = 2 * _K * (_GROUPS - _SG) * 16
    for _ in range(_K):
        pltpu.make_async_copy(
            table.at[pl.ds(0, (_GROUPS - _SG) * 16)],
            o_out.at[pl.ds(0, (_GROUPS - _SG) * 16)],
            sem_w,
        ).wait()
        pltpu.make_async_copy(
            table.at[pl.ds(0, (_GROUPS - _SG) * 16)],
            d_out.at[pl.ds(0, (_GROUPS - _SG) * 16)],
            sem_w,
        ).wait()


@jax.jit
def kernel(ori, dest, table):
    return _od_gather(table, ori, dest)
